# baseline (speedup 1.0000x reference)
import functools

import jax
import jax.numpy as jnp
from jax import lax
from jax.experimental import pallas as pl
from jax.experimental.pallas import tpu as pltpu
from jax.experimental.pallas import tpu_sc as plsc

_B = 16384
_EMB = 64
_EPS = 1e-5
_N = 1000000

_BL = 8192
_NBLK = -(-_N // _BL)
_NPK = _NBLK * (_BL // 2)
_NB128 = -(-_N // 128)


def _repack_body(t_ref, eye_ref, out_ref):
    y = t_ref[...]
    eye = eye_ref[...]
    h = _BL // 2
    dn = (((0,), (0,)), ((), ()))
    out_ref[:, :_EMB] = lax.dot_general(
        y[:, :h], eye, dn, preferred_element_type=jnp.float32)
    out_ref[:, _EMB:] = lax.dot_general(
        y[:, h:], eye, dn, preferred_element_type=jnp.float32)


@functools.lru_cache(maxsize=None)
def _make_repack():
    return pl.pallas_call(
        _repack_body,
        grid=(_NBLK,),
        in_specs=[
            pl.BlockSpec((_EMB, _BL), lambda i: (0, i)),
            pl.BlockSpec((_EMB, _EMB), lambda i: (0, 0)),
        ],
        out_specs=pl.BlockSpec((_BL // 2, 128), lambda i: (i, 0)),
        out_shape=jax.ShapeDtypeStruct((_NPK, 128), jnp.float32),
        compiler_params=pltpu.CompilerParams(fuse_transposed_lhs_in_matmul=True),
    )


@functools.lru_cache(maxsize=None)
def _make_sc_gather():
    info = plsc.get_sparse_core_info()
    nc, ns = info.num_cores, info.num_subcores
    nw = nc * ns
    bpw = _B // nw
    ec = 256
    bc = 128
    f32 = jnp.float32
    i32 = jnp.int32

    mesh = plsc.VectorSubcoreMesh(core_axis_name="c", subcore_axis_name="s")

    def body(uid_hbm, iid_hbm, upk_hbm, ipk_hbm,
             ue_out, ie_out,
             uid_v, iid_v, urow_v, irow_v,
             ue_v, ie_v,
             s0, s1):
        wid = lax.axis_index("s") * nc + lax.axis_index("c")
        base = wid * bpw
        pltpu.sync_copy(uid_hbm.at[pl.ds(base, bpw)], uid_v)
        pltpu.sync_copy(iid_hbm.at[pl.ds(base, bpw)], iid_v)
        for k in range(bpw // 16):
            sl = pl.ds(k * 16, 16)
            u = uid_v[sl]
            i = iid_v[sl]
            urow_v[sl] = ((u >> 13) << 12) + (u & 4095)
            irow_v[sl] = i >> 1
        for h in range(bpw // ec):
            cu = pltpu.async_copy(
                upk_hbm.at[urow_v.at[pl.ds(h * ec, ec)]], ue_v, s0)
            ci = pltpu.async_copy(
                ipk_hbm.at[irow_v.at[pl.ds(h * ec, ec)]], ie_v, s1)
            cu.wait()
            pltpu.sync_copy(ue_v, ue_out.at[pl.ds(base + h * ec, ec)])
            ci.wait()
            pltpu.sync_copy(ie_v, ie_out.at[pl.ds(base + h * ec, ec)])

    return pl.kernel(
        body,
        out_type=(
            jax.ShapeDtypeStruct((_B, 128), f32),
            jax.ShapeDtypeStruct((_B, 128), f32),
        ),
        mesh=mesh,
        compiler_params=pltpu.CompilerParams(needs_layout_passes=False),
        scratch_types=[
            pltpu.VMEM((bpw,), i32),
            pltpu.VMEM((bpw,), i32),
            pltpu.VMEM((bpw,), i32),
            pltpu.VMEM((bpw,), i32),
            pltpu.VMEM((ec, 128), f32),
            pltpu.VMEM((ec, 128), f32),
            pltpu.SemaphoreType.DMA,
            pltpu.SemaphoreType.DMA,
        ],
    )


@functools.lru_cache(maxsize=None)
def _make_sc_bias():
    info = plsc.get_sparse_core_info()
    nc, ns = info.num_cores, info.num_subcores
    nw = nc * ns
    bpw = _B // nw
    f32 = jnp.float32
    i32 = jnp.int32

    mesh = plsc.VectorSubcoreMesh(core_axis_name="c", subcore_axis_name="s")

    def body(uid_hbm, iid_hbm, ub_hbm, ib_hbm, bias_out,
             uid_v, iid_v, uhi_v, ihi_v, ubr_v, ibr_v, bsum_v, s2, s3):
        wid = lax.axis_index("s") * nc + lax.axis_index("c")
        base = wid * bpw
        pltpu.sync_copy(uid_hbm.at[pl.ds(base, bpw)], uid_v)
        pltpu.sync_copy(iid_hbm.at[pl.ds(base, bpw)], iid_v)
        for k in range(bpw // 16):
            sl = pl.ds(k * 16, 16)
            uhi_v[sl] = uid_v[sl] >> 4
            ihi_v[sl] = iid_v[sl] >> 4
        cb0 = pltpu.async_copy(ub_hbm.at[uhi_v], ubr_v, s2)
        cb1 = pltpu.async_copy(ib_hbm.at[ihi_v], ibr_v, s3)
        cb0.wait()
        cb1.wait()
        rid = lax.iota(i32, 16)
        for k in range(bpw // 16):
            sl = pl.ds(k * 16, 16)
            r = rid + (k * 16)
            bu = plsc.load_gather(ubr_v, [r, uid_v[sl] & 15])
            bi = plsc.load_gather(ibr_v, [r, iid_v[sl] & 15])
            bsum_v[sl] = bu + bi
        pltpu.sync_copy(bsum_v, bias_out.at[pl.ds(base, bpw)])

    return pl.kernel(
        body,
        out_type=jax.ShapeDtypeStruct((_B,), f32),
        mesh=mesh,
        compiler_params=pltpu.CompilerParams(
            use_tc_tiling_on_sc=False, needs_layout_passes=False),
        scratch_types=[
            pltpu.VMEM((bpw,), i32),
            pltpu.VMEM((bpw,), i32),
            pltpu.VMEM((bpw,), i32),
            pltpu.VMEM((bpw,), i32),
            pltpu.VMEM((bpw, 16), f32),
            pltpu.VMEM((bpw, 16), f32),
            pltpu.VMEM((bpw,), f32),
            pltpu.SemaphoreType.DMA,
            pltpu.SemaphoreType.DMA,
        ],
    )


_BM = 2048


def _mlp_body(ue_ref, ie_ref, uh_ref, ih_ref, bias_ref,
              w1a_ref, w1b_ref, b1_ref, s1_ref, be1_ref,
              w2_ref, b2_ref, s2_ref, be2_ref,
              w3_ref, b3_ref, s3_ref, be3_ref,
              w4_ref, b4_ref, out_ref):
    f32 = jnp.float32
    ue128 = ue_ref[...]
    ie128 = ie_ref[...]
    ue = jnp.where(uh_ref[...] > 0, ue128[:, _EMB:], ue128[:, :_EMB])
    ie = jnp.where(ih_ref[...] > 0, ie128[:, _EMB:], ie128[:, :_EMB])
    h = jnp.dot(ue, w1a_ref[...], preferred_element_type=f32)
    h += jnp.dot(ie, w1b_ref[...], preferred_element_type=f32)
    h = jnp.maximum(h + b1_ref[...], 0.0) * s1_ref[...] + be1_ref[...]
    h = jnp.dot(h, w2_ref[...], preferred_element_type=f32)
    h = jnp.maximum(h + b2_ref[...], 0.0) * s2_ref[...] + be2_ref[...]
    h = jnp.dot(h, w3_ref[...], preferred_element_type=f32)
    h = jnp.maximum(h + b3_ref[...], 0.0) * s3_ref[...] + be3_ref[...]
    out = jnp.sum(h * w4_ref[...], axis=1)
    out_ref[...] = out + b4_ref[0, 0] + bias_ref[...]


@functools.lru_cache(maxsize=None)
def _make_mlp():
    f32 = jnp.float32
    bspec_batch = lambda w: pl.BlockSpec((_BM, w), lambda i: (i, 0))
    bspec_full = lambda r, c: pl.BlockSpec((r, c), lambda i: (0, 0))
    in_specs = [
        bspec_batch(128),
        bspec_batch(128),
        bspec_batch(1),
        bspec_batch(1),
        pl.BlockSpec((_BM,), lambda i: (i,)),
        bspec_full(_EMB, 256),
        bspec_full(_EMB, 256),
        bspec_full(1, 256),
        bspec_full(1, 256),
        bspec_full(1, 256),
        bspec_full(256, 128),
        bspec_full(1, 128),
        bspec_full(1, 128),
        bspec_full(1, 128),
        bspec_full(128, 64),
        bspec_full(1, 64),
        bspec_full(1, 64),
        bspec_full(1, 64),
        bspec_full(1, 64),
        bspec_full(1, 1),
    ]
    return pl.pallas_call(
        _mlp_body,
        grid=(_B // _BM,),
        in_specs=in_specs,
        out_specs=pl.BlockSpec((_BM,), lambda i: (i,)),
        out_shape=jax.ShapeDtypeStruct((_B,), f32),
    )


def kernel(user_ids, item_ids, user_emb, item_emb, user_bias, item_bias,
           W1, b1, g1, be1, W2, b2, g2, be2, W3, b3, g3, be3, W4, b4):
    uid = user_ids.astype(jnp.int32)
    iid = item_ids.astype(jnp.int32)
    uh = ((uid >> 12) & 1).reshape(-1, 1)
    ih = (iid & 1).reshape(-1, 1)

    repack = _make_repack()
    eye = jnp.eye(_EMB, dtype=jnp.float32)
    upk = repack(user_emb.T, eye)
    ipk = item_emb.reshape(_N // 2, 128)
    ub16 = user_bias.reshape(-1, 16)
    ib16 = item_bias.reshape(-1, 16)

    bias = _make_sc_bias()(uid, iid, ub16, ib16)
    ue, ie = _make_sc_gather()(uid, iid, upk, ipk)

    inv = jnp.float32(1.0) / jnp.sqrt(jnp.float32(1.0 + _EPS))
    row = lambda v: v.reshape(1, -1)
    out = _make_mlp()(
        ue, ie, uh, ih, bias,
        W1[:_EMB], W1[_EMB:], row(b1), row(g1 * inv), row(be1),
        W2, row(b2), row(g2 * inv), row(be2),
        W3, row(b3), row(g3 * inv), row(be3),
        W4.reshape(1, -1), b4.reshape(1, 1),
    )
    return out


# speedup vs baseline: 1.4327x; 1.4327x over previous
import functools

import jax
import jax.numpy as jnp
from jax import lax
from jax.experimental import pallas as pl
from jax.experimental.pallas import tpu as pltpu
from jax.experimental.pallas import tpu_sc as plsc

_B = 16384
_EMB = 64
_EPS = 1e-5
_N = 1000000

_BL = 16384
_NBLK = -(-_N // _BL)
_NPK = _NBLK * (_BL // 2)
_NB128 = -(-_N // 128)


def _repack_body(t_ref, eye_ref, out_ref):
    y = t_ref[...]
    eye = eye_ref[...]
    h = _BL // 2
    dn = (((0,), (0,)), ((), ()))
    out_ref[:, :_EMB] = lax.dot_general(
        y[:, :h], eye, dn, preferred_element_type=jnp.float32)
    out_ref[:, _EMB:] = lax.dot_general(
        y[:, h:], eye, dn, preferred_element_type=jnp.float32)


@functools.lru_cache(maxsize=None)
def _make_repack():
    return pl.pallas_call(
        _repack_body,
        grid=(_NBLK,),
        in_specs=[
            pl.BlockSpec((_EMB, _BL), lambda i: (0, i)),
            pl.BlockSpec((_EMB, _EMB), lambda i: (0, 0)),
        ],
        out_specs=pl.BlockSpec((_BL // 2, 128), lambda i: (i, 0)),
        out_shape=jax.ShapeDtypeStruct((_NPK, 128), jnp.float32),
        compiler_params=pltpu.CompilerParams(fuse_transposed_lhs_in_matmul=True),
    )


@functools.lru_cache(maxsize=None)
def _make_sc_gather():
    info = plsc.get_sparse_core_info()
    nc, ns = info.num_cores, info.num_subcores
    nw = nc * ns
    bpw = _B // nw
    ec = 256
    bc = 128
    f32 = jnp.float32
    i32 = jnp.int32

    mesh = plsc.VectorSubcoreMesh(core_axis_name="c", subcore_axis_name="s")

    def body(uid_hbm, iid_hbm, upk_hbm, ipk_hbm,
             ue_out, ie_out,
             uid_v, iid_v, urow_v, irow_v,
             ue_v, ie_v,
             s0, s1):
        wid = lax.axis_index("s") * nc + lax.axis_index("c")
        base = wid * bpw
        pltpu.sync_copy(uid_hbm.at[pl.ds(base, bpw)], uid_v)
        pltpu.sync_copy(iid_hbm.at[pl.ds(base, bpw)], iid_v)
        for k in range(bpw // 16):
            sl = pl.ds(k * 16, 16)
            u = uid_v[sl]
            i = iid_v[sl]
            urow_v[sl] = ((u >> 14) << 13) + (u & 8191)
            irow_v[sl] = ((i >> 14) << 13) + (i & 8191)
        for h in range(bpw // ec):
            cu = pltpu.async_copy(
                upk_hbm.at[urow_v.at[pl.ds(h * ec, ec)]], ue_v, s0)
            ci = pltpu.async_copy(
                ipk_hbm.at[irow_v.at[pl.ds(h * ec, ec)]], ie_v, s1)
            cu.wait()
            pltpu.sync_copy(ue_v, ue_out.at[pl.ds(base + h * ec, ec)])
            ci.wait()
            pltpu.sync_copy(ie_v, ie_out.at[pl.ds(base + h * ec, ec)])

    return pl.kernel(
        body,
        out_type=(
            jax.ShapeDtypeStruct((_B, 128), f32),
            jax.ShapeDtypeStruct((_B, 128), f32),
        ),
        mesh=mesh,
        compiler_params=pltpu.CompilerParams(needs_layout_passes=False),
        scratch_types=[
            pltpu.VMEM((bpw,), i32),
            pltpu.VMEM((bpw,), i32),
            pltpu.VMEM((bpw,), i32),
            pltpu.VMEM((bpw,), i32),
            pltpu.VMEM((ec, 128), f32),
            pltpu.VMEM((ec, 128), f32),
            pltpu.SemaphoreType.DMA,
            pltpu.SemaphoreType.DMA,
        ],
    )


@functools.lru_cache(maxsize=None)
def _make_sc_bias():
    info = plsc.get_sparse_core_info()
    nc, ns = info.num_cores, info.num_subcores
    nw = nc * ns
    bpw = _B // nw
    f32 = jnp.float32
    i32 = jnp.int32

    mesh = plsc.VectorSubcoreMesh(core_axis_name="c", subcore_axis_name="s")

    def body(uid_hbm, iid_hbm, ub_hbm, ib_hbm, bias_out,
             uid_v, iid_v, uhi_v, ihi_v, ubr_v, ibr_v, bsum_v, s2, s3):
        wid = lax.axis_index("s") * nc + lax.axis_index("c")
        base = wid * bpw
        pltpu.sync_copy(uid_hbm.at[pl.ds(base, bpw)], uid_v)
        pltpu.sync_copy(iid_hbm.at[pl.ds(base, bpw)], iid_v)
        for k in range(bpw // 16):
            sl = pl.ds(k * 16, 16)
            uhi_v[sl] = uid_v[sl] >> 4
            ihi_v[sl] = iid_v[sl] >> 4
        cb0 = pltpu.async_copy(ub_hbm.at[uhi_v], ubr_v, s2)
        cb1 = pltpu.async_copy(ib_hbm.at[ihi_v], ibr_v, s3)
        cb0.wait()
        cb1.wait()
        rid = lax.iota(i32, 16)
        for k in range(bpw // 16):
            sl = pl.ds(k * 16, 16)
            r = rid + (k * 16)
            bu = plsc.load_gather(ubr_v, [r, uid_v[sl] & 15])
            bi = plsc.load_gather(ibr_v, [r, iid_v[sl] & 15])
            bsum_v[sl] = bu + bi
        pltpu.sync_copy(bsum_v, bias_out.at[pl.ds(base, bpw)])

    return pl.kernel(
        body,
        out_type=jax.ShapeDtypeStruct((_B,), f32),
        mesh=mesh,
        compiler_params=pltpu.CompilerParams(
            use_tc_tiling_on_sc=False, needs_layout_passes=False),
        scratch_types=[
            pltpu.VMEM((bpw,), i32),
            pltpu.VMEM((bpw,), i32),
            pltpu.VMEM((bpw,), i32),
            pltpu.VMEM((bpw,), i32),
            pltpu.VMEM((bpw, 16), f32),
            pltpu.VMEM((bpw, 16), f32),
            pltpu.VMEM((bpw,), f32),
            pltpu.SemaphoreType.DMA,
            pltpu.SemaphoreType.DMA,
        ],
    )


_BM = 2048


def _mlp_body(ue_ref, ie_ref, uh_ref, ih_ref, bias_ref,
              w1a_ref, w1b_ref, b1_ref, s1_ref, be1_ref,
              w2_ref, b2_ref, s2_ref, be2_ref,
              w3_ref, b3_ref, s3_ref, be3_ref,
              w4_ref, b4_ref, out_ref):
    f32 = jnp.float32
    ue128 = ue_ref[...]
    ie128 = ie_ref[...]
    ue = jnp.where(uh_ref[...] > 0, ue128[:, _EMB:], ue128[:, :_EMB])
    ie = jnp.where(ih_ref[...] > 0, ie128[:, _EMB:], ie128[:, :_EMB])
    h = jnp.dot(ue, w1a_ref[...], preferred_element_type=f32)
    h += jnp.dot(ie, w1b_ref[...], preferred_element_type=f32)
    h = jnp.maximum(h + b1_ref[...], 0.0) * s1_ref[...] + be1_ref[...]
    h = jnp.dot(h, w2_ref[...], preferred_element_type=f32)
    h = jnp.maximum(h + b2_ref[...], 0.0) * s2_ref[...] + be2_ref[...]
    h = jnp.dot(h, w3_ref[...], preferred_element_type=f32)
    h = jnp.maximum(h + b3_ref[...], 0.0) * s3_ref[...] + be3_ref[...]
    out = jnp.sum(h * w4_ref[...], axis=1)
    out_ref[...] = out + b4_ref[0, 0] + bias_ref[...]


@functools.lru_cache(maxsize=None)
def _make_mlp():
    f32 = jnp.float32
    bspec_batch = lambda w: pl.BlockSpec((_BM, w), lambda i: (i, 0))
    bspec_full = lambda r, c: pl.BlockSpec((r, c), lambda i: (0, 0))
    in_specs = [
        bspec_batch(128),
        bspec_batch(128),
        bspec_batch(1),
        bspec_batch(1),
        pl.BlockSpec((_BM,), lambda i: (i,)),
        bspec_full(_EMB, 256),
        bspec_full(_EMB, 256),
        bspec_full(1, 256),
        bspec_full(1, 256),
        bspec_full(1, 256),
        bspec_full(256, 128),
        bspec_full(1, 128),
        bspec_full(1, 128),
        bspec_full(1, 128),
        bspec_full(128, 64),
        bspec_full(1, 64),
        bspec_full(1, 64),
        bspec_full(1, 64),
        bspec_full(1, 64),
        bspec_full(1, 1),
    ]
    return pl.pallas_call(
        _mlp_body,
        grid=(_B // _BM,),
        in_specs=in_specs,
        out_specs=pl.BlockSpec((_BM,), lambda i: (i,)),
        out_shape=jax.ShapeDtypeStruct((_B,), f32),
    )


def kernel(user_ids, item_ids, user_emb, item_emb, user_bias, item_bias,
           W1, b1, g1, be1, W2, b2, g2, be2, W3, b3, g3, be3, W4, b4):
    uid = user_ids.astype(jnp.int32)
    iid = item_ids.astype(jnp.int32)
    uh = ((uid >> 13) & 1).reshape(-1, 1)
    ih = ((iid >> 13) & 1).reshape(-1, 1)

    repack = _make_repack()
    eye = jnp.eye(_EMB, dtype=jnp.float32)
    upk = repack(user_emb.T, eye)
    ipk = repack(item_emb.T, eye)
    ub16 = user_bias.T.reshape(-1, 16)
    ib16 = item_bias.T.reshape(-1, 16)

    bias = _make_sc_bias()(uid, iid, ub16, ib16)
    ue, ie = _make_sc_gather()(uid, iid, upk, ipk)

    inv = jnp.float32(1.0) / jnp.sqrt(jnp.float32(1.0 + _EPS))
    row = lambda v: v.reshape(1, -1)
    out = _make_mlp()(
        ue, ie, uh, ih, bias,
        W1[:_EMB], W1[_EMB:], row(b1), row(g1 * inv), row(be1),
        W2, row(b2), row(g2 * inv), row(be2),
        W3, row(b3), row(g3 * inv), row(be3),
        W4.reshape(1, -1), b4.reshape(1, 1),
    )
    return out


# speedup vs baseline: 1.5001x; 1.0471x over previous
import functools

import jax
import jax.numpy as jnp
from jax import lax
from jax.experimental import pallas as pl
from jax.experimental.pallas import tpu as pltpu
from jax.experimental.pallas import tpu_sc as plsc

_B = 16384
_EMB = 64
_EPS = 1e-5
_N = 1000000

_BL = 32768
_NBLK = -(-_N // _BL)
_NPK = _NBLK * (_BL // 2)
_NB128 = -(-_N // 128)


def _repack_body(t_ref, eye_ref, out_ref):
    y = t_ref[...]
    eye = eye_ref[...]
    h = _BL // 2
    dn = (((0,), (0,)), ((), ()))
    out_ref[:, :_EMB] = lax.dot_general(
        y[:, :h], eye, dn, preferred_element_type=jnp.float32)
    out_ref[:, _EMB:] = lax.dot_general(
        y[:, h:], eye, dn, preferred_element_type=jnp.float32)


@functools.lru_cache(maxsize=None)
def _make_repack():
    return pl.pallas_call(
        _repack_body,
        grid=(_NBLK,),
        in_specs=[
            pl.BlockSpec((_EMB, _BL), lambda i: (0, i)),
            pl.BlockSpec((_EMB, _EMB), lambda i: (0, 0)),
        ],
        out_specs=pl.BlockSpec((_BL // 2, 128), lambda i: (i, 0)),
        out_shape=jax.ShapeDtypeStruct((_NPK, 128), jnp.float32),
        compiler_params=pltpu.CompilerParams(fuse_transposed_lhs_in_matmul=True),
    )


@functools.lru_cache(maxsize=None)
def _make_sc_gather():
    info = plsc.get_sparse_core_info()
    nc, ns = info.num_cores, info.num_subcores
    nw = nc * ns
    bpw = _B // nw
    ec = 256
    bc = 128
    f32 = jnp.float32
    i32 = jnp.int32

    mesh = plsc.VectorSubcoreMesh(core_axis_name="c", subcore_axis_name="s")

    def body(uid_hbm, iid_hbm, upk_hbm, ipk_hbm,
             ue_out, ie_out,
             uid_v, iid_v, urow_v, irow_v,
             ue_v, ie_v,
             s0, s1):
        wid = lax.axis_index("s") * nc + lax.axis_index("c")
        base = wid * bpw
        pltpu.sync_copy(uid_hbm.at[pl.ds(base, bpw)], uid_v)
        pltpu.sync_copy(iid_hbm.at[pl.ds(base, bpw)], iid_v)
        for k in range(bpw // 16):
            sl = pl.ds(k * 16, 16)
            u = uid_v[sl]
            i = iid_v[sl]
            urow_v[sl] = ((u >> 15) << 14) + (u & 16383)
            irow_v[sl] = ((i >> 15) << 14) + (i & 16383)
        for h in range(bpw // ec):
            cu = pltpu.async_copy(
                upk_hbm.at[urow_v.at[pl.ds(h * ec, ec)]], ue_v, s0)
            ci = pltpu.async_copy(
                ipk_hbm.at[irow_v.at[pl.ds(h * ec, ec)]], ie_v, s1)
            cu.wait()
            pltpu.sync_copy(ue_v, ue_out.at[pl.ds(base + h * ec, ec)])
            ci.wait()
            pltpu.sync_copy(ie_v, ie_out.at[pl.ds(base + h * ec, ec)])

    return pl.kernel(
        body,
        out_type=(
            jax.ShapeDtypeStruct((_B, 128), f32),
            jax.ShapeDtypeStruct((_B, 128), f32),
        ),
        mesh=mesh,
        compiler_params=pltpu.CompilerParams(needs_layout_passes=False),
        scratch_types=[
            pltpu.VMEM((bpw,), i32),
            pltpu.VMEM((bpw,), i32),
            pltpu.VMEM((bpw,), i32),
            pltpu.VMEM((bpw,), i32),
            pltpu.VMEM((ec, 128), f32),
            pltpu.VMEM((ec, 128), f32),
            pltpu.SemaphoreType.DMA,
            pltpu.SemaphoreType.DMA,
        ],
    )


@functools.lru_cache(maxsize=None)
def _make_sc_bias():
    info = plsc.get_sparse_core_info()
    nc, ns = info.num_cores, info.num_subcores
    nw = nc * ns
    bpw = _B // nw
    f32 = jnp.float32
    i32 = jnp.int32

    mesh = plsc.VectorSubcoreMesh(core_axis_name="c", subcore_axis_name="s")

    def body(uid_hbm, iid_hbm, ub_hbm, ib_hbm, bias_out,
             uid_v, iid_v, uhi_v, ihi_v, ubr_v, ibr_v, bsum_v, s2, s3):
        wid = lax.axis_index("s") * nc + lax.axis_index("c")
        base = wid * bpw
        pltpu.sync_copy(uid_hbm.at[pl.ds(base, bpw)], uid_v)
        pltpu.sync_copy(iid_hbm.at[pl.ds(base, bpw)], iid_v)
        for k in range(bpw // 16):
            sl = pl.ds(k * 16, 16)
            uhi_v[sl] = uid_v[sl] >> 4
            ihi_v[sl] = iid_v[sl] >> 4
        cb0 = pltpu.async_copy(ub_hbm.at[uhi_v], ubr_v, s2)
        cb1 = pltpu.async_copy(ib_hbm.at[ihi_v], ibr_v, s3)
        cb0.wait()
        cb1.wait()
        rid = lax.iota(i32, 16)
        for k in range(bpw // 16):
            sl = pl.ds(k * 16, 16)
            r = rid + (k * 16)
            bu = plsc.load_gather(ubr_v, [r, uid_v[sl] & 15])
            bi = plsc.load_gather(ibr_v, [r, iid_v[sl] & 15])
            bsum_v[sl] = bu + bi
        pltpu.sync_copy(bsum_v, bias_out.at[pl.ds(base, bpw)])

    return pl.kernel(
        body,
        out_type=jax.ShapeDtypeStruct((_B,), f32),
        mesh=mesh,
        compiler_params=pltpu.CompilerParams(
            use_tc_tiling_on_sc=False, needs_layout_passes=False),
        scratch_types=[
            pltpu.VMEM((bpw,), i32),
            pltpu.VMEM((bpw,), i32),
            pltpu.VMEM((bpw,), i32),
            pltpu.VMEM((bpw,), i32),
            pltpu.VMEM((bpw, 16), f32),
            pltpu.VMEM((bpw, 16), f32),
            pltpu.VMEM((bpw,), f32),
            pltpu.SemaphoreType.DMA,
            pltpu.SemaphoreType.DMA,
        ],
    )


_BM = 2048


def _mlp_body(ue_ref, ie_ref, uh_ref, ih_ref, bias_ref,
              w1a_ref, w1b_ref, b1_ref, s1_ref, be1_ref,
              w2_ref, b2_ref, s2_ref, be2_ref,
              w3_ref, b3_ref, s3_ref, be3_ref,
              w4_ref, b4_ref, out_ref):
    f32 = jnp.float32
    ue128 = ue_ref[...]
    ie128 = ie_ref[...]
    ue = jnp.where(uh_ref[...] > 0, ue128[:, _EMB:], ue128[:, :_EMB])
    ie = jnp.where(ih_ref[...] > 0, ie128[:, _EMB:], ie128[:, :_EMB])
    h = jnp.dot(ue, w1a_ref[...], preferred_element_type=f32)
    h += jnp.dot(ie, w1b_ref[...], preferred_element_type=f32)
    h = jnp.maximum(h + b1_ref[...], 0.0) * s1_ref[...] + be1_ref[...]
    h = jnp.dot(h, w2_ref[...], preferred_element_type=f32)
    h = jnp.maximum(h + b2_ref[...], 0.0) * s2_ref[...] + be2_ref[...]
    h = jnp.dot(h, w3_ref[...], preferred_element_type=f32)
    h = jnp.maximum(h + b3_ref[...], 0.0) * s3_ref[...] + be3_ref[...]
    out = jnp.sum(h * w4_ref[...], axis=1)
    out_ref[...] = out + b4_ref[0, 0] + bias_ref[...]


@functools.lru_cache(maxsize=None)
def _make_mlp():
    f32 = jnp.float32
    bspec_batch = lambda w: pl.BlockSpec((_BM, w), lambda i: (i, 0))
    bspec_full = lambda r, c: pl.BlockSpec((r, c), lambda i: (0, 0))
    in_specs = [
        bspec_batch(128),
        bspec_batch(128),
        bspec_batch(1),
        bspec_batch(1),
        pl.BlockSpec((_BM,), lambda i: (i,)),
        bspec_full(_EMB, 256),
        bspec_full(_EMB, 256),
        bspec_full(1, 256),
        bspec_full(1, 256),
        bspec_full(1, 256),
        bspec_full(256, 128),
        bspec_full(1, 128),
        bspec_full(1, 128),
        bspec_full(1, 128),
        bspec_full(128, 64),
        bspec_full(1, 64),
        bspec_full(1, 64),
        bspec_full(1, 64),
        bspec_full(1, 64),
        bspec_full(1, 1),
    ]
    return pl.pallas_call(
        _mlp_body,
        grid=(_B // _BM,),
        in_specs=in_specs,
        out_specs=pl.BlockSpec((_BM,), lambda i: (i,)),
        out_shape=jax.ShapeDtypeStruct((_B,), f32),
    )


def kernel(user_ids, item_ids, user_emb, item_emb, user_bias, item_bias,
           W1, b1, g1, be1, W2, b2, g2, be2, W3, b3, g3, be3, W4, b4):
    uid = user_ids.astype(jnp.int32)
    iid = item_ids.astype(jnp.int32)
    uh = ((uid >> 14) & 1).reshape(-1, 1)
    ih = ((iid >> 14) & 1).reshape(-1, 1)

    repack = _make_repack()
    eye = jnp.eye(_EMB, dtype=jnp.float32)
    upk = repack(user_emb.T, eye)
    ipk = repack(item_emb.T, eye)
    ub16 = user_bias.T.reshape(-1, 16)
    ib16 = item_bias.T.reshape(-1, 16)

    bias = _make_sc_bias()(uid, iid, ub16, ib16)
    ue, ie = _make_sc_gather()(uid, iid, upk, ipk)

    inv = jnp.float32(1.0) / jnp.sqrt(jnp.float32(1.0 + _EPS))
    row = lambda v: v.reshape(1, -1)
    out = _make_mlp()(
        ue, ie, uh, ih, bias,
        W1[:_EMB], W1[_EMB:], row(b1), row(g1 * inv), row(be1),
        W2, row(b2), row(g2 * inv), row(be2),
        W3, row(b3), row(g3 * inv), row(be3),
        W4.reshape(1, -1), b4.reshape(1, 1),
    )
    return out


# speedup vs baseline: 1.6785x; 1.1189x over previous
import functools

import jax
import jax.numpy as jnp
from jax import lax
from jax.experimental import pallas as pl
from jax.experimental.pallas import tpu as pltpu
from jax.experimental.pallas import tpu_sc as plsc

_B = 16384
_EMB = 64
_EPS = 1e-5
_N = 1000000

_BL = 32768
_NBLK = -(-_N // _BL)
_NPK = _NBLK * (_BL // 2)
_NB128 = -(-_N // 128)


def _repack_body(t_ref, eye_ref, out_ref):
    y = t_ref[...]
    eye = eye_ref[...]
    h = _BL // 2
    dn = (((0,), (0,)), ((), ()))
    out_ref[:, :_EMB] = lax.dot_general(
        y[:, :h], eye, dn, preferred_element_type=jnp.float32)
    out_ref[:, _EMB:] = lax.dot_general(
        y[:, h:], eye, dn, preferred_element_type=jnp.float32)


@functools.lru_cache(maxsize=None)
def _make_repack():
    return pl.pallas_call(
        _repack_body,
        grid=(_NBLK,),
        in_specs=[
            pl.BlockSpec((_EMB, _BL), lambda i: (0, i)),
            pl.BlockSpec((_EMB, _EMB), lambda i: (0, 0)),
        ],
        out_specs=pl.BlockSpec((_BL // 2, 128), lambda i: (i, 0)),
        out_shape=jax.ShapeDtypeStruct((_NPK, 128), jnp.float32),
        compiler_params=pltpu.CompilerParams(fuse_transposed_lhs_in_matmul=True),
    )


@functools.lru_cache(maxsize=None)
def _make_sc_gather():
    info = plsc.get_sparse_core_info()
    nc, ns = info.num_cores, info.num_subcores
    nw = nc * ns
    bpw = _B // nw
    ec = 256
    bc = 128
    f32 = jnp.float32
    i32 = jnp.int32

    mesh = plsc.VectorSubcoreMesh(core_axis_name="c", subcore_axis_name="s")

    def body(uid_hbm, iid_hbm, upk_hbm, ipk_hbm,
             ue_out, ie_out,
             uid_v, iid_v, urow_v, irow_v,
             ue_v, ie_v,
             s0, s1):
        wid = lax.axis_index("s") * nc + lax.axis_index("c")
        base = wid * bpw
        pltpu.sync_copy(uid_hbm.at[pl.ds(base, bpw)], uid_v)
        pltpu.sync_copy(iid_hbm.at[pl.ds(base, bpw)], iid_v)
        for k in range(bpw // 16):
            sl = pl.ds(k * 16, 16)
            u = uid_v[sl]
            i = iid_v[sl]
            urow_v[sl] = ((u >> 15) << 14) + (u & 16383)
            irow_v[sl] = ((i >> 15) << 14) + (i & 16383)
        for h in range(bpw // ec):
            cu = pltpu.async_copy(
                upk_hbm.at[urow_v.at[pl.ds(h * ec, ec)]], ue_v, s0)
            ci = pltpu.async_copy(
                ipk_hbm.at[irow_v.at[pl.ds(h * ec, ec)]], ie_v, s1)
            cu.wait()
            pltpu.sync_copy(ue_v, ue_out.at[pl.ds(base + h * ec, ec)])
            ci.wait()
            pltpu.sync_copy(ie_v, ie_out.at[pl.ds(base + h * ec, ec)])

    return pl.kernel(
        body,
        out_type=(
            jax.ShapeDtypeStruct((_B, 128), f32),
            jax.ShapeDtypeStruct((_B, 128), f32),
        ),
        mesh=mesh,
        compiler_params=pltpu.CompilerParams(needs_layout_passes=False),
        scratch_types=[
            pltpu.VMEM((bpw,), i32),
            pltpu.VMEM((bpw,), i32),
            pltpu.VMEM((bpw,), i32),
            pltpu.VMEM((bpw,), i32),
            pltpu.VMEM((ec, 128), f32),
            pltpu.VMEM((ec, 128), f32),
            pltpu.SemaphoreType.DMA,
            pltpu.SemaphoreType.DMA,
        ],
    )


_BBLK = 65536
_NBB = -(-_N // _BBLK)


def _bsq_body(x_ref, out_ref):
    out_ref[...] = x_ref[...].reshape(_BBLK // 128, 128)


@functools.lru_cache(maxsize=None)
def _make_bias_squeeze():
    return pl.pallas_call(
        _bsq_body,
        grid=(_NBB,),
        in_specs=[pl.BlockSpec((1, _BBLK), lambda i: (0, i))],
        out_specs=pl.BlockSpec((_BBLK // 128, 128), lambda i: (i, 0)),
        out_shape=jax.ShapeDtypeStruct((_NBB * _BBLK // 128, 128), jnp.float32),
    )


@functools.lru_cache(maxsize=None)
def _make_sc_bias():
    info = plsc.get_sparse_core_info()
    nc, ns = info.num_cores, info.num_subcores
    nw = nc * ns
    bpw = _B // nw
    bc = 256
    f32 = jnp.float32
    i32 = jnp.int32

    mesh = plsc.VectorSubcoreMesh(core_axis_name="c", subcore_axis_name="s")

    def body(uid_hbm, iid_hbm, ub_hbm, ib_hbm, bias_out,
             uid_v, iid_v, uhi_v, ihi_v, ubr_v, ibr_v, bsum_v, s2, s3):
        wid = lax.axis_index("s") * nc + lax.axis_index("c")
        base = wid * bpw
        pltpu.sync_copy(uid_hbm.at[pl.ds(base, bpw)], uid_v)
        pltpu.sync_copy(iid_hbm.at[pl.ds(base, bpw)], iid_v)
        for k in range(bpw // 16):
            sl = pl.ds(k * 16, 16)
            uhi_v[sl] = uid_v[sl] >> 7
            ihi_v[sl] = iid_v[sl] >> 7
        rid = lax.iota(i32, 16)
        for c in range(bpw // bc):
            cb0 = pltpu.async_copy(
                ub_hbm.at[uhi_v.at[pl.ds(c * bc, bc)]], ubr_v, s2)
            cb1 = pltpu.async_copy(
                ib_hbm.at[ihi_v.at[pl.ds(c * bc, bc)]], ibr_v, s3)
            cb0.wait()
            cb1.wait()
            for k in range(bc // 16):
                sl = pl.ds(c * bc + k * 16, 16)
                r = rid + (k * 16)
                bu = plsc.load_gather(ubr_v, [r, uid_v[sl] & 127])
                bi = plsc.load_gather(ibr_v, [r, iid_v[sl] & 127])
                bsum_v[sl] = bu + bi
        pltpu.sync_copy(bsum_v, bias_out.at[pl.ds(base, bpw)])

    return pl.kernel(
        body,
        out_type=jax.ShapeDtypeStruct((_B,), f32),
        mesh=mesh,
        compiler_params=pltpu.CompilerParams(
            use_tc_tiling_on_sc=False, needs_layout_passes=False),
        scratch_types=[
            pltpu.VMEM((bpw,), i32),
            pltpu.VMEM((bpw,), i32),
            pltpu.VMEM((bpw,), i32),
            pltpu.VMEM((bpw,), i32),
            pltpu.VMEM((bc, 128), f32),
            pltpu.VMEM((bc, 128), f32),
            pltpu.VMEM((bpw,), f32),
            pltpu.SemaphoreType.DMA,
            pltpu.SemaphoreType.DMA,
        ],
    )


_BM = 2048


def _mlp_body(ue_ref, ie_ref, uh_ref, ih_ref, bias_ref,
              w1a_ref, w1b_ref, b1_ref, s1_ref, be1_ref,
              w2_ref, b2_ref, s2_ref, be2_ref,
              w3_ref, b3_ref, s3_ref, be3_ref,
              w4_ref, b4_ref, out_ref):
    f32 = jnp.float32
    ue128 = ue_ref[...]
    ie128 = ie_ref[...]
    ue = jnp.where(uh_ref[...] > 0, ue128[:, _EMB:], ue128[:, :_EMB])
    ie = jnp.where(ih_ref[...] > 0, ie128[:, _EMB:], ie128[:, :_EMB])
    h = jnp.dot(ue, w1a_ref[...], preferred_element_type=f32)
    h += jnp.dot(ie, w1b_ref[...], preferred_element_type=f32)
    h = jnp.maximum(h + b1_ref[...], 0.0) * s1_ref[...] + be1_ref[...]
    h = jnp.dot(h, w2_ref[...], preferred_element_type=f32)
    h = jnp.maximum(h + b2_ref[...], 0.0) * s2_ref[...] + be2_ref[...]
    h = jnp.dot(h, w3_ref[...], preferred_element_type=f32)
    h = jnp.maximum(h + b3_ref[...], 0.0) * s3_ref[...] + be3_ref[...]
    out = jnp.sum(h * w4_ref[...], axis=1)
    out_ref[...] = out + b4_ref[0, 0] + bias_ref[...]


@functools.lru_cache(maxsize=None)
def _make_mlp():
    f32 = jnp.float32
    bspec_batch = lambda w: pl.BlockSpec((_BM, w), lambda i: (i, 0))
    bspec_full = lambda r, c: pl.BlockSpec((r, c), lambda i: (0, 0))
    in_specs = [
        bspec_batch(128),
        bspec_batch(128),
        bspec_batch(1),
        bspec_batch(1),
        pl.BlockSpec((_BM,), lambda i: (i,)),
        bspec_full(_EMB, 256),
        bspec_full(_EMB, 256),
        bspec_full(1, 256),
        bspec_full(1, 256),
        bspec_full(1, 256),
        bspec_full(256, 128),
        bspec_full(1, 128),
        bspec_full(1, 128),
        bspec_full(1, 128),
        bspec_full(128, 64),
        bspec_full(1, 64),
        bspec_full(1, 64),
        bspec_full(1, 64),
        bspec_full(1, 64),
        bspec_full(1, 1),
    ]
    return pl.pallas_call(
        _mlp_body,
        grid=(_B // _BM,),
        in_specs=in_specs,
        out_specs=pl.BlockSpec((_BM,), lambda i: (i,)),
        out_shape=jax.ShapeDtypeStruct((_B,), f32),
    )


def kernel(user_ids, item_ids, user_emb, item_emb, user_bias, item_bias,
           W1, b1, g1, be1, W2, b2, g2, be2, W3, b3, g3, be3, W4, b4):
    uid = user_ids.astype(jnp.int32)
    iid = item_ids.astype(jnp.int32)
    uh = ((uid >> 14) & 1).reshape(-1, 1)
    ih = ((iid >> 14) & 1).reshape(-1, 1)

    repack = _make_repack()
    eye = jnp.eye(_EMB, dtype=jnp.float32)
    upk = repack(user_emb.T, eye)
    ipk = repack(item_emb.T, eye)
    bsq = _make_bias_squeeze()
    ub128 = bsq(user_bias.T)
    ib128 = bsq(item_bias.T)

    bias = _make_sc_bias()(uid, iid, ub128, ib128)
    ue, ie = _make_sc_gather()(uid, iid, upk, ipk)

    inv = jnp.float32(1.0) / jnp.sqrt(jnp.float32(1.0 + _EPS))
    row = lambda v: v.reshape(1, -1)
    out = _make_mlp()(
        ue, ie, uh, ih, bias,
        W1[:_EMB], W1[_EMB:], row(b1), row(g1 * inv), row(be1),
        W2, row(b2), row(g2 * inv), row(be2),
        W3, row(b3), row(g3 * inv), row(be3),
        W4.reshape(1, -1), b4.reshape(1, 1),
    )
    return out
